# baseline (speedup 1.0000x reference)
import dataclasses
import functools

import jax
import jax.numpy as jnp
from jax import lax
from jax.experimental import pallas as pl
from jax.experimental.pallas import tpu as pltpu
from jax.experimental.pallas import tpu_sc as plsc

N_NODES = 10000
N_EDGES = 320000
ROW_BLK = 2000
N_GRID = N_NODES // ROW_BLK

LANES = 16
N_TILES = 32
E_PAD = 327680
SCAN_BLKS = 80
TSLICE = 313
ACC_R = 320
TRASH = TSLICE
CAP = 10752
GCHUNK = 80
PBITS = 14


def _sc_seg_sums(table, edges, tq, mq, q_chunks):
    mesh = plsc.VectorSubcoreMesh(core_axis_name="c", subcore_axis_name="s")
    cp = pltpu.CompilerParams()
    if "needs_layout_passes" in pltpu.CompilerParams.__dataclass_fields__:
        cp = dataclasses.replace(cp, needs_layout_passes=False)

    @functools.partial(
        pl.kernel,
        mesh=mesh,
        compiler_params=cp,
        out_type=jax.ShapeDtypeStruct((q_chunks * N_TILES * ACC_R, 256),
                                      jnp.float32),
        scratch_types=[
            pltpu.VMEM((64, 64), jnp.int32),
            pltpu.VMEM((64, 64), jnp.int32),
            pltpu.VMEM((CAP,), jnp.int32),
            pltpu.VMEM((GCHUNK,), jnp.int32),
            pltpu.VMEM((GCHUNK,), jnp.int32),
            pltpu.VMEM((GCHUNK, 128), jnp.float32),
            pltpu.VMEM((GCHUNK, 128), jnp.float32),
            pltpu.VMEM((ACC_R, 256), jnp.float32),
            pltpu.VMEM((2, 128), jnp.float32),
            pltpu.SemaphoreType.DMA,
            pltpu.SemaphoreType.DMA,
        ],
    )
    def k(table_h, eb_h, t_h, m_h, out_h,
          scan0, scan1, elist, sb0, sb1, rows0, rows1, acc, tm_v, sem0, sem1):
        c = lax.axis_index("c")
        s = lax.axis_index("s")
        tid = c * 16 + s
        base = tid * TSLICE

        tl = jnp.full((LANES,), TRASH << PBITS, jnp.int32)

        @pl.loop(0, CAP // LANES)
        def _(i):
            elist[pl.ds(i * LANES, LANES)] = tl

        lanes = lax.iota(jnp.int32, LANES)

        def scan_block(buf, cnt):
            def row_body(r, cnt):
                for gi in range(4):
                    sl = pl.ds(gi * LANES, LANES)
                    sv = buf[r, sl]
                    dv = buf[r + 32, sl]
                    lo = dv - base
                    ok = (lo >= 0) & (lo < TSLICE)
                    oki = jnp.where(ok, 1, 0).astype(jnp.int32)
                    inc = plsc.cumsum(oki)
                    p = jnp.where(ok, cnt + inc - oki, CAP - LANES + lanes)
                    plsc.store_scatter(elist, [p], sv + (lo << PBITS))
                    cnt = jnp.minimum(cnt + inc[LANES - 1], CAP - 2 * GCHUNK)
                return cnt

            return lax.fori_loop(0, 32, row_body, cnt)

        pltpu.async_copy(eb_h.at[pl.ds(0, 64)], scan0, sem0)

        def blk_pair(b, cnt):
            pltpu.async_copy(eb_h.at[pl.ds((2 * b + 1) * 64, 64)], scan1,
                             sem1)
            pltpu.make_async_copy(eb_h.at[pl.ds(0, 64)], scan0, sem0).wait()
            cnt = scan_block(scan0, cnt)

            @pl.when(b < SCAN_BLKS - 1)
            def _():
                pltpu.async_copy(eb_h.at[pl.ds((2 * b + 2) * 64, 64)], scan0,
                                 sem0)

            pltpu.make_async_copy(eb_h.at[pl.ds(0, 64)], scan1, sem1).wait()
            return scan_block(scan1, cnt)

        cnt = lax.fori_loop(0, SCAN_BLKS, blk_pair, jnp.int32(0))
        n_pairs = (cnt + 2 * GCHUNK - 1) // (2 * GCHUNK)

        @pl.loop(0, q_chunks)
        def _pass(q):
            pltpu.sync_copy(t_h.at[pl.ds(q, 1)], tm_v.at[pl.ds(0, 1)])
            pltpu.sync_copy(m_h.at[pl.ds(q, 1)], tm_v.at[pl.ds(1, 1)])
            zf = jnp.zeros((LANES,), jnp.float32)

            @pl.loop(0, ACC_R)
            def _(r):
                for g in range(256 // LANES):
                    acc[r, pl.ds(g * LANES, LANES)] = zf

            qoff = q * N_NODES
            smask = (1 << PBITS) - 1

            def _unpack(ch, sb):
                @pl.loop(0, GCHUNK // LANES)
                def _(g):
                    sl = pl.ds(g * LANES, LANES)
                    e16 = elist[pl.ds(ch * GCHUNK + g * LANES, LANES)]
                    sb[sl] = (e16 & smask) + qoff

            def _compute(ch, rows):
                ts = [tm_v[0, pl.ds(g * LANES, LANES)] for g in range(8)]
                ms = [tm_v[1, pl.ds(g * LANES, LANES)] for g in range(8)]

                @plsc.parallel_loop(0, GCHUNK // LANES, unroll=2)
                def _(sub):
                    lv = elist[pl.ds(ch * GCHUNK + sub * LANES, LANES)]
                    for j in range(LANES):
                        r = lv[j] >> PBITS
                        for g in range(8):
                            sl = pl.ds(g * LANES, LANES)
                            v = rows[sub * LANES + j, sl]
                            e = jnp.exp(v * ts[g] - ms[g])
                            plsc.addupdate(acc.at[r, sl], e)
                            plsc.addupdate(
                                acc.at[r, pl.ds(128 + g * LANES, LANES)],
                                e * v)

            _unpack(0, sb0)
            pltpu.async_copy(table_h.at[sb0], rows0, sem0)

            @pl.loop(0, n_pairs)
            def _(i):
                ch = i * 2
                _unpack(ch + 1, sb1)
                pltpu.async_copy(table_h.at[sb1], rows1, sem1)
                pltpu.make_async_copy(table_h.at[sb0], rows0, sem0).wait()
                _compute(ch, rows0)

                @pl.when(i < n_pairs - 1)
                def _():
                    _unpack(ch + 2, sb0)
                    pltpu.async_copy(table_h.at[sb0], rows0, sem0)

                pltpu.make_async_copy(table_h.at[sb1], rows1, sem1).wait()
                _compute(ch + 1, rows1)

            row0 = q * N_TILES * ACC_R + tid * ACC_R
            pltpu.sync_copy(acc, out_h.at[pl.ds(row0, ACC_R)])

    return k(table, edges, tq, mq)


def _sc_aggr(table, edges, t, m, q_chunks):
    o = _sc_seg_sums(table, edges, t.reshape(q_chunks, 128),
                     m.reshape(q_chunks, 128), q_chunks)
    o = o.reshape(q_chunks, N_TILES, ACC_R, 256)
    o = o[:, :, :TSLICE, :].reshape(q_chunks, N_TILES * TSLICE, 256)
    o = o[:, :N_NODES, :]
    return o[:, :, 128:], o[:, :, :128]


def _colmax_body(x_ref, t_ref, m_ref):
    i = pl.program_id(0)
    mx = jnp.max(x_ref[...] * t_ref[...], axis=0, keepdims=True)

    @pl.when(i == 0)
    def _():
        m_ref[...] = mx

    @pl.when(i > 0)
    def _():
        m_ref[...] = jnp.maximum(m_ref[...], mx)


def _colmax(x, t):
    n, d = x.shape
    return pl.pallas_call(
        _colmax_body,
        grid=(N_GRID,),
        in_specs=[
            pl.BlockSpec((ROW_BLK, d), lambda i: (i, 0)),
            pl.BlockSpec((1, d), lambda i: (0, 0)),
        ],
        out_specs=pl.BlockSpec((1, d), lambda i: (0, 0)),
        out_shape=jax.ShapeDtypeStruct((1, d), jnp.float32),
    )(x, t)


def _sage_dense_body(q_chunks, num_ref, den_ref, x_ref, wl_ref, wr_ref,
                     y_ref, s1_ref, s2_ref):
    i = pl.program_id(0)
    y = jnp.dot(x_ref[...], wr_ref[...], preferred_element_type=jnp.float32)
    for q in range(q_chunks):
        aggr = num_ref[q] / (den_ref[q] + 1e-16)
        y += jnp.dot(aggr, wl_ref[q], preferred_element_type=jnp.float32)
    y_ref[...] = y
    s1 = jnp.sum(y).reshape(1, 1)
    s2 = jnp.sum(y * y).reshape(1, 1)

    @pl.when(i == 0)
    def _():
        s1_ref[...] = s1
        s2_ref[...] = s2

    @pl.when(i > 0)
    def _():
        s1_ref[...] += s1
        s2_ref[...] += s2


def _sage_dense(num, den, x, wlt, wrt):
    n, d = x.shape
    q_chunks = num.shape[0]
    h = wrt.shape[1]
    wlq = wlt.reshape(q_chunks, 128, h)
    return pl.pallas_call(
        functools.partial(_sage_dense_body, q_chunks),
        grid=(N_GRID,),
        in_specs=[
            pl.BlockSpec((q_chunks, ROW_BLK, 128), lambda i: (0, i, 0)),
            pl.BlockSpec((q_chunks, ROW_BLK, 128), lambda i: (0, i, 0)),
            pl.BlockSpec((ROW_BLK, d), lambda i: (i, 0)),
            pl.BlockSpec((q_chunks, 128, h), lambda i: (0, 0, 0)),
            pl.BlockSpec((d, h), lambda i: (0, 0)),
        ],
        out_specs=[
            pl.BlockSpec((ROW_BLK, h), lambda i: (i, 0)),
            pl.BlockSpec((1, 1), lambda i: (0, 0)),
            pl.BlockSpec((1, 1), lambda i: (0, 0)),
        ],
        out_shape=[
            jax.ShapeDtypeStruct((n, h), jnp.float32),
            jax.ShapeDtypeStruct((1, 1), jnp.float32),
            jax.ShapeDtypeStruct((1, 1), jnp.float32),
        ],
    )(num, den, x, wlq, wrt)


def _norm_relu_body(n_elems, y_ref, s1_ref, s2_ref, w_ref, b_ref, t_ref,
                    h_ref, m_ref, c_ref):
    i = pl.program_id(0)
    mu = s1_ref[0, 0] / n_elems
    var = jnp.maximum(s2_ref[0, 0] / n_elems - mu * mu, 0.0)
    inv = 1.0 / (jnp.sqrt(var) + 1e-5)
    h = jnp.maximum((y_ref[...] - mu) * inv * w_ref[...] + b_ref[...], 0.0)
    h_ref[...] = h
    for q in range(c_ref.shape[0]):
        c_ref[q] = h[:, q * 128:(q + 1) * 128]
    mx = jnp.max(h * t_ref[...], axis=0, keepdims=True)

    @pl.when(i == 0)
    def _():
        m_ref[...] = mx

    @pl.when(i > 0)
    def _():
        m_ref[...] = jnp.maximum(m_ref[...], mx)


def _norm_relu(y, s1, s2, w, b, t):
    n, h = y.shape
    q_chunks = h // 128
    return pl.pallas_call(
        functools.partial(_norm_relu_body, float(n * h)),
        grid=(N_GRID,),
        in_specs=[
            pl.BlockSpec((ROW_BLK, h), lambda i: (i, 0)),
            pl.BlockSpec((1, 1), lambda i: (0, 0)),
            pl.BlockSpec((1, 1), lambda i: (0, 0)),
            pl.BlockSpec((1, h), lambda i: (0, 0)),
            pl.BlockSpec((1, h), lambda i: (0, 0)),
            pl.BlockSpec((1, h), lambda i: (0, 0)),
        ],
        out_specs=[
            pl.BlockSpec((ROW_BLK, h), lambda i: (i, 0)),
            pl.BlockSpec((1, h), lambda i: (0, 0)),
            pl.BlockSpec((q_chunks, ROW_BLK, 128), lambda i: (0, i, 0)),
        ],
        out_shape=[
            jax.ShapeDtypeStruct((n, h), jnp.float32),
            jax.ShapeDtypeStruct((1, h), jnp.float32),
            jax.ShapeDtypeStruct((q_chunks, n, 128), jnp.float32),
        ],
    )(y, s1, s2, w, b, t)


def _final_body(n_elems, y_ref, s1_ref, s2_ref, w_ref, b_ref, mlw_ref, fxw_ref,
                fxb_ref, out_ref, acc_ref):
    i = pl.program_id(0)
    mu = s1_ref[0, 0] / n_elems
    var = jnp.maximum(s2_ref[0, 0] / n_elems - mu * mu, 0.0)
    inv = 1.0 / (jnp.sqrt(var) + 1e-5)
    h = jnp.maximum((y_ref[...] - mu) * inv * w_ref[...] + b_ref[...], 0.0)
    cs = jnp.sum(h, axis=0, keepdims=True)

    @pl.when(i == 0)
    def _():
        acc_ref[...] = cs

    @pl.when(i > 0)
    def _():
        acc_ref[...] += cs

    @pl.when(i == pl.num_programs(0) - 1)
    def _():
        pooled = jnp.dot(acc_ref[...], mlw_ref[...],
                         preferred_element_type=jnp.float32)
        out_ref[...] = jnp.dot(pooled, fxw_ref[...],
                               preferred_element_type=jnp.float32) + fxb_ref[...]


def _final(y, s1, s2, w, b, mlwt, fxwt, fxb):
    n, h = y.shape
    return pl.pallas_call(
        functools.partial(_final_body, float(n * h)),
        grid=(N_GRID,),
        in_specs=[
            pl.BlockSpec((ROW_BLK, h), lambda i: (i, 0)),
            pl.BlockSpec((1, 1), lambda i: (0, 0)),
            pl.BlockSpec((1, 1), lambda i: (0, 0)),
            pl.BlockSpec((1, h), lambda i: (0, 0)),
            pl.BlockSpec((1, h), lambda i: (0, 0)),
            pl.BlockSpec(mlwt.shape, lambda i: (0, 0)),
            pl.BlockSpec(fxwt.shape, lambda i: (0, 0)),
            pl.BlockSpec((1, fxwt.shape[1]), lambda i: (0, 0)),
        ],
        out_specs=pl.BlockSpec((1, fxwt.shape[1]), lambda i: (0, 0)),
        out_shape=jax.ShapeDtypeStruct((1, fxwt.shape[1]), jnp.float32),
        scratch_shapes=[pltpu.VMEM((1, h), jnp.float32)],
    )(y, s1, s2, w, b, mlwt, fxwt, fxb)


def kernel(x, edge_index, t1, W1l, W1r, ln1_w, ln1_b, t2, W2l, W2r, ln2_w,
           ln2_b, mem_k, mem_conv_w, mem_lin_w, fx_w, fx_b):
    src = edge_index[0]
    dst = edge_index[1]
    pad = E_PAD - N_EDGES
    src_p = jnp.concatenate(
        [src, jnp.zeros((pad,), jnp.int32)]).reshape(SCAN_BLKS * 2, 32, 64)
    dst_p = jnp.concatenate(
        [dst, jnp.full((pad,), -1, jnp.int32)]).reshape(SCAN_BLKS * 2, 32, 64)
    edges = jnp.concatenate([src_p, dst_p], axis=1).reshape(-1, 64)

    m1 = _colmax(x, t1)
    num1, den1 = _sc_aggr(x, edges, t1, m1, 1)
    y1, s1a, s1b = _sage_dense(num1, den1, x, W1l.T, W1r.T)
    h1, m2, h1c = _norm_relu(y1, s1a, s1b, ln1_w.reshape(1, -1),
                             ln1_b.reshape(1, -1), t2)

    num2, den2 = _sc_aggr(h1c.reshape(-1, 128), edges, t2, m2, 4)
    y2, s2a, s2b = _sage_dense(num2, den2, h1, W2l.T, W2r.T)

    return _final(y2, s2a, s2b, ln2_w.reshape(1, -1), ln2_b.reshape(1, -1),
                  mem_lin_w.T, fx_w.T, fx_b.reshape(1, -1))


# speedup vs baseline: 1.0785x; 1.0785x over previous
import dataclasses
import functools

import jax
import jax.numpy as jnp
from jax import lax
from jax.experimental import pallas as pl
from jax.experimental.pallas import tpu as pltpu
from jax.experimental.pallas import tpu_sc as plsc

N_NODES = 10000
N_EDGES = 320000
ROW_BLK = 2000
N_GRID = N_NODES // ROW_BLK

LANES = 16
N_TILES = 32
E_PAD = 327680
SCAN_BLKS = 80
TSLICE = 313
ACC_R = 320
TRASH = TSLICE
CAP = 10752
GCHUNK = 48
PBITS = 14


def _sc_seg_sums(table, edges, tq, mq, q_chunks):
    mesh = plsc.VectorSubcoreMesh(core_axis_name="c", subcore_axis_name="s")
    cp = pltpu.CompilerParams()
    if "needs_layout_passes" in pltpu.CompilerParams.__dataclass_fields__:
        cp = dataclasses.replace(cp, needs_layout_passes=False)

    @functools.partial(
        pl.kernel,
        mesh=mesh,
        compiler_params=cp,
        out_type=jax.ShapeDtypeStruct((q_chunks * N_TILES * ACC_R, 256),
                                      jnp.float32),
        scratch_types=[
            pltpu.VMEM((64, 64), jnp.int32),
            pltpu.VMEM((64, 64), jnp.int32),
            pltpu.VMEM((CAP,), jnp.int32),
            pltpu.VMEM((GCHUNK,), jnp.int32),
            pltpu.VMEM((GCHUNK,), jnp.int32),
            pltpu.VMEM((GCHUNK, 128), jnp.float32),
            pltpu.VMEM((GCHUNK, 128), jnp.float32),
            pltpu.VMEM((ACC_R, 256), jnp.float32),
            pltpu.VMEM((2, 128), jnp.float32),
            pltpu.SemaphoreType.DMA,
            pltpu.SemaphoreType.DMA,
        ],
    )
    def k(table_h, eb_h, t_h, m_h, out_h,
          scan0, scan1, elist, sb0, sb1, rows0, rows1, acc, tm_v, sem0, sem1):
        c = lax.axis_index("c")
        s = lax.axis_index("s")
        tid = c * 16 + s
        base = tid * TSLICE

        tl = jnp.full((LANES,), TRASH << PBITS, jnp.int32)

        @pl.loop(0, CAP // LANES)
        def _(i):
            elist[pl.ds(i * LANES, LANES)] = tl

        lanes = lax.iota(jnp.int32, LANES)

        def scan_block(buf, cnt):
            def row_body(r, cnt):
                for gi in range(4):
                    sl = pl.ds(gi * LANES, LANES)
                    sv = buf[r, sl]
                    dv = buf[r + 32, sl]
                    lo = dv - base
                    ok = (lo >= 0) & (lo < TSLICE)
                    oki = jnp.where(ok, 1, 0).astype(jnp.int32)
                    inc = plsc.cumsum(oki)
                    p = jnp.where(ok, cnt + inc - oki, CAP - LANES + lanes)
                    plsc.store_scatter(elist, [p], sv + (lo << PBITS))
                    cnt = jnp.minimum(cnt + inc[LANES - 1], CAP - 2 * GCHUNK)
                return cnt

            return lax.fori_loop(0, 32, row_body, cnt)

        pltpu.async_copy(eb_h.at[pl.ds(0, 64)], scan0, sem0)

        def blk_pair(b, cnt):
            pltpu.async_copy(eb_h.at[pl.ds((2 * b + 1) * 64, 64)], scan1,
                             sem1)
            pltpu.make_async_copy(eb_h.at[pl.ds(0, 64)], scan0, sem0).wait()
            cnt = scan_block(scan0, cnt)

            @pl.when(b < SCAN_BLKS - 1)
            def _():
                pltpu.async_copy(eb_h.at[pl.ds((2 * b + 2) * 64, 64)], scan0,
                                 sem0)

            pltpu.make_async_copy(eb_h.at[pl.ds(0, 64)], scan1, sem1).wait()
            return scan_block(scan1, cnt)

        cnt = lax.fori_loop(0, SCAN_BLKS, blk_pair, jnp.int32(0))
        n_pairs = (cnt + 2 * GCHUNK - 1) // (2 * GCHUNK)

        @pl.loop(0, q_chunks)
        def _pass(q):
            pltpu.sync_copy(t_h.at[pl.ds(q, 1)], tm_v.at[pl.ds(0, 1)])
            pltpu.sync_copy(m_h.at[pl.ds(q, 1)], tm_v.at[pl.ds(1, 1)])
            zf = jnp.zeros((LANES,), jnp.float32)

            @pl.loop(0, ACC_R)
            def _(r):
                for g in range(256 // LANES):
                    acc[r, pl.ds(g * LANES, LANES)] = zf

            qoff = q * N_NODES
            smask = (1 << PBITS) - 1

            def _unpack(ch, sb):
                @pl.loop(0, GCHUNK // LANES)
                def _(g):
                    sl = pl.ds(g * LANES, LANES)
                    e16 = elist[pl.ds(ch * GCHUNK + g * LANES, LANES)]
                    sb[sl] = (e16 & smask) + qoff

            def _compute(ch, rows):
                ts = [tm_v[0, pl.ds(g * LANES, LANES)] for g in range(8)]
                ms = [tm_v[1, pl.ds(g * LANES, LANES)] for g in range(8)]
                for sub in range(GCHUNK // LANES):
                    lv = elist[pl.ds(ch * GCHUNK + sub * LANES, LANES)]
                    for j in range(LANES):
                        r = lv[j] >> PBITS
                        for g in range(8):
                            sl = pl.ds(g * LANES, LANES)
                            v = rows[sub * LANES + j, sl]
                            e = jnp.exp(v * ts[g] - ms[g])
                            plsc.addupdate(acc.at[r, sl], e)
                            plsc.addupdate(
                                acc.at[r, pl.ds(128 + g * LANES, LANES)],
                                e * v)

            _unpack(0, sb0)
            pltpu.async_copy(table_h.at[sb0], rows0, sem0)

            @pl.loop(0, n_pairs)
            def _(i):
                ch = i * 2
                _unpack(ch + 1, sb1)
                pltpu.async_copy(table_h.at[sb1], rows1, sem1)
                pltpu.make_async_copy(table_h.at[sb0], rows0, sem0).wait()
                _compute(ch, rows0)

                @pl.when(i < n_pairs - 1)
                def _():
                    _unpack(ch + 2, sb0)
                    pltpu.async_copy(table_h.at[sb0], rows0, sem0)

                pltpu.make_async_copy(table_h.at[sb1], rows1, sem1).wait()
                _compute(ch + 1, rows1)

            row0 = q * N_TILES * ACC_R + tid * ACC_R
            pltpu.sync_copy(acc, out_h.at[pl.ds(row0, ACC_R)])

    return k(table, edges, tq, mq)


def _sc_aggr(table, edges, t, m, q_chunks):
    o = _sc_seg_sums(table, edges, t.reshape(q_chunks, 128),
                     m.reshape(q_chunks, 128), q_chunks)
    o = o.reshape(q_chunks, N_TILES, ACC_R, 256)
    o = o[:, :, :TSLICE, :].reshape(q_chunks, N_TILES * TSLICE, 256)
    o = o[:, :N_NODES, :]
    return o[:, :, 128:], o[:, :, :128]


def _colmax_body(x_ref, t_ref, m_ref):
    i = pl.program_id(0)
    mx = jnp.max(x_ref[...] * t_ref[...], axis=0, keepdims=True)

    @pl.when(i == 0)
    def _():
        m_ref[...] = mx

    @pl.when(i > 0)
    def _():
        m_ref[...] = jnp.maximum(m_ref[...], mx)


def _colmax(x, t):
    n, d = x.shape
    return pl.pallas_call(
        _colmax_body,
        grid=(N_GRID,),
        in_specs=[
            pl.BlockSpec((ROW_BLK, d), lambda i: (i, 0)),
            pl.BlockSpec((1, d), lambda i: (0, 0)),
        ],
        out_specs=pl.BlockSpec((1, d), lambda i: (0, 0)),
        out_shape=jax.ShapeDtypeStruct((1, d), jnp.float32),
    )(x, t)


def _sage_dense_body(q_chunks, num_ref, den_ref, x_ref, wl_ref, wr_ref,
                     y_ref, s1_ref, s2_ref):
    i = pl.program_id(0)
    y = jnp.dot(x_ref[...], wr_ref[...], preferred_element_type=jnp.float32)
    for q in range(q_chunks):
        aggr = num_ref[q] / (den_ref[q] + 1e-16)
        y += jnp.dot(aggr, wl_ref[q], preferred_element_type=jnp.float32)
    y_ref[...] = y
    s1 = jnp.sum(y).reshape(1, 1)
    s2 = jnp.sum(y * y).reshape(1, 1)

    @pl.when(i == 0)
    def _():
        s1_ref[...] = s1
        s2_ref[...] = s2

    @pl.when(i > 0)
    def _():
        s1_ref[...] += s1
        s2_ref[...] += s2


def _sage_dense(num, den, x, wlt, wrt):
    n, d = x.shape
    q_chunks = num.shape[0]
    h = wrt.shape[1]
    wlq = wlt.reshape(q_chunks, 128, h)
    return pl.pallas_call(
        functools.partial(_sage_dense_body, q_chunks),
        grid=(N_GRID,),
        in_specs=[
            pl.BlockSpec((q_chunks, ROW_BLK, 128), lambda i: (0, i, 0)),
            pl.BlockSpec((q_chunks, ROW_BLK, 128), lambda i: (0, i, 0)),
            pl.BlockSpec((ROW_BLK, d), lambda i: (i, 0)),
            pl.BlockSpec((q_chunks, 128, h), lambda i: (0, 0, 0)),
            pl.BlockSpec((d, h), lambda i: (0, 0)),
        ],
        out_specs=[
            pl.BlockSpec((ROW_BLK, h), lambda i: (i, 0)),
            pl.BlockSpec((1, 1), lambda i: (0, 0)),
            pl.BlockSpec((1, 1), lambda i: (0, 0)),
        ],
        out_shape=[
            jax.ShapeDtypeStruct((n, h), jnp.float32),
            jax.ShapeDtypeStruct((1, 1), jnp.float32),
            jax.ShapeDtypeStruct((1, 1), jnp.float32),
        ],
    )(num, den, x, wlq, wrt)


def _norm_relu_body(n_elems, y_ref, s1_ref, s2_ref, w_ref, b_ref, t_ref,
                    h_ref, m_ref, c_ref):
    i = pl.program_id(0)
    mu = s1_ref[0, 0] / n_elems
    var = jnp.maximum(s2_ref[0, 0] / n_elems - mu * mu, 0.0)
    inv = 1.0 / (jnp.sqrt(var) + 1e-5)
    h = jnp.maximum((y_ref[...] - mu) * inv * w_ref[...] + b_ref[...], 0.0)
    h_ref[...] = h
    for q in range(c_ref.shape[0]):
        c_ref[q] = h[:, q * 128:(q + 1) * 128]
    mx = jnp.max(h * t_ref[...], axis=0, keepdims=True)

    @pl.when(i == 0)
    def _():
        m_ref[...] = mx

    @pl.when(i > 0)
    def _():
        m_ref[...] = jnp.maximum(m_ref[...], mx)


def _norm_relu(y, s1, s2, w, b, t):
    n, h = y.shape
    q_chunks = h // 128
    return pl.pallas_call(
        functools.partial(_norm_relu_body, float(n * h)),
        grid=(N_GRID,),
        in_specs=[
            pl.BlockSpec((ROW_BLK, h), lambda i: (i, 0)),
            pl.BlockSpec((1, 1), lambda i: (0, 0)),
            pl.BlockSpec((1, 1), lambda i: (0, 0)),
            pl.BlockSpec((1, h), lambda i: (0, 0)),
            pl.BlockSpec((1, h), lambda i: (0, 0)),
            pl.BlockSpec((1, h), lambda i: (0, 0)),
        ],
        out_specs=[
            pl.BlockSpec((ROW_BLK, h), lambda i: (i, 0)),
            pl.BlockSpec((1, h), lambda i: (0, 0)),
            pl.BlockSpec((q_chunks, ROW_BLK, 128), lambda i: (0, i, 0)),
        ],
        out_shape=[
            jax.ShapeDtypeStruct((n, h), jnp.float32),
            jax.ShapeDtypeStruct((1, h), jnp.float32),
            jax.ShapeDtypeStruct((q_chunks, n, 128), jnp.float32),
        ],
    )(y, s1, s2, w, b, t)


def _final_body(n_elems, y_ref, s1_ref, s2_ref, w_ref, b_ref, mlw_ref, fxw_ref,
                fxb_ref, out_ref, acc_ref):
    i = pl.program_id(0)
    mu = s1_ref[0, 0] / n_elems
    var = jnp.maximum(s2_ref[0, 0] / n_elems - mu * mu, 0.0)
    inv = 1.0 / (jnp.sqrt(var) + 1e-5)
    h = jnp.maximum((y_ref[...] - mu) * inv * w_ref[...] + b_ref[...], 0.0)
    cs = jnp.sum(h, axis=0, keepdims=True)

    @pl.when(i == 0)
    def _():
        acc_ref[...] = cs

    @pl.when(i > 0)
    def _():
        acc_ref[...] += cs

    @pl.when(i == pl.num_programs(0) - 1)
    def _():
        pooled = jnp.dot(acc_ref[...], mlw_ref[...],
                         preferred_element_type=jnp.float32)
        out_ref[...] = jnp.dot(pooled, fxw_ref[...],
                               preferred_element_type=jnp.float32) + fxb_ref[...]


def _final(y, s1, s2, w, b, mlwt, fxwt, fxb):
    n, h = y.shape
    return pl.pallas_call(
        functools.partial(_final_body, float(n * h)),
        grid=(N_GRID,),
        in_specs=[
            pl.BlockSpec((ROW_BLK, h), lambda i: (i, 0)),
            pl.BlockSpec((1, 1), lambda i: (0, 0)),
            pl.BlockSpec((1, 1), lambda i: (0, 0)),
            pl.BlockSpec((1, h), lambda i: (0, 0)),
            pl.BlockSpec((1, h), lambda i: (0, 0)),
            pl.BlockSpec(mlwt.shape, lambda i: (0, 0)),
            pl.BlockSpec(fxwt.shape, lambda i: (0, 0)),
            pl.BlockSpec((1, fxwt.shape[1]), lambda i: (0, 0)),
        ],
        out_specs=pl.BlockSpec((1, fxwt.shape[1]), lambda i: (0, 0)),
        out_shape=jax.ShapeDtypeStruct((1, fxwt.shape[1]), jnp.float32),
        scratch_shapes=[pltpu.VMEM((1, h), jnp.float32)],
    )(y, s1, s2, w, b, mlwt, fxwt, fxb)


def kernel(x, edge_index, t1, W1l, W1r, ln1_w, ln1_b, t2, W2l, W2r, ln2_w,
           ln2_b, mem_k, mem_conv_w, mem_lin_w, fx_w, fx_b):
    src = edge_index[0]
    dst = edge_index[1]
    pad = E_PAD - N_EDGES
    src_p = jnp.concatenate(
        [src, jnp.zeros((pad,), jnp.int32)]).reshape(SCAN_BLKS * 2, 32, 64)
    dst_p = jnp.concatenate(
        [dst, jnp.full((pad,), -1, jnp.int32)]).reshape(SCAN_BLKS * 2, 32, 64)
    edges = jnp.concatenate([src_p, dst_p], axis=1).reshape(-1, 64)

    m1 = _colmax(x, t1)
    num1, den1 = _sc_aggr(x, edges, t1, m1, 1)
    y1, s1a, s1b = _sage_dense(num1, den1, x, W1l.T, W1r.T)
    h1, m2, h1c = _norm_relu(y1, s1a, s1b, ln1_w.reshape(1, -1),
                             ln1_b.reshape(1, -1), t2)

    num2, den2 = _sc_aggr(h1c.reshape(-1, 128), edges, t2, m2, 4)
    y2, s2a, s2b = _sage_dense(num2, den2, h1, W2l.T, W2r.T)

    return _final(y2, s2a, s2b, ln2_w.reshape(1, -1), ln2_b.reshape(1, -1),
                  mem_lin_w.T, fx_w.T, fx_b.reshape(1, -1))


# speedup vs baseline: 1.4335x; 1.3292x over previous
import dataclasses
import functools

import jax
import jax.numpy as jnp
from jax import lax
from jax.experimental import pallas as pl
from jax.experimental.pallas import tpu as pltpu
from jax.experimental.pallas import tpu_sc as plsc

N_NODES = 10000
N_EDGES = 320000
ROW_BLK = 2000
N_GRID = N_NODES // ROW_BLK

LANES = 16
N_TILES = 32
E_PAD = 327680
SCAN_BLKS = 80
TSLICE = 313
ACC_R = 320
TRASH = TSLICE
CAP = 10752
GCHUNK = 96
PBITS = 14


def _sc_seg_sums(table, edges, tq, mq, q_chunks):
    mesh = plsc.VectorSubcoreMesh(core_axis_name="c", subcore_axis_name="s")
    cp = pltpu.CompilerParams()
    if "needs_layout_passes" in pltpu.CompilerParams.__dataclass_fields__:
        cp = dataclasses.replace(cp, needs_layout_passes=False)

    @functools.partial(
        pl.kernel,
        mesh=mesh,
        compiler_params=cp,
        out_type=jax.ShapeDtypeStruct((q_chunks * N_TILES * ACC_R, 256),
                                      jnp.float32),
        scratch_types=[
            pltpu.VMEM((64, 64), jnp.int32),
            pltpu.VMEM((CAP,), jnp.int32),
            pltpu.VMEM((GCHUNK,), jnp.int32),
            pltpu.VMEM((GCHUNK,), jnp.int32),
            pltpu.VMEM((GCHUNK, 128), jnp.float32),
            pltpu.VMEM((GCHUNK, 128), jnp.float32),
            pltpu.VMEM((ACC_R, 256), jnp.float32),
            pltpu.VMEM((2, 128), jnp.float32),
            pltpu.SemaphoreType.DMA,
            pltpu.SemaphoreType.DMA,
        ],
    )
    def k(table_h, eb_h, t_h, m_h, out_h,
          scan0, elist, sb0, sb1, rows0, rows1, acc, tm_v, sem0, sem1):
        c = lax.axis_index("c")
        s = lax.axis_index("s")
        tid = c * 16 + s
        base = tid * TSLICE

        tl = jnp.full((LANES,), TRASH << PBITS, jnp.int32)

        @pl.loop(0, CAP // LANES)
        def _(i):
            elist[pl.ds(i * LANES, LANES)] = tl

        lanes = lax.iota(jnp.int32, LANES)

        def scan_block(buf, cnt):
            def row_body(r, cnt):
                for gi in range(4):
                    sl = pl.ds(gi * LANES, LANES)
                    sv = buf[r, sl]
                    dv = buf[r + 32, sl]
                    lo = dv - base
                    ok = (lo >= 0) & (lo < TSLICE)
                    oki = jnp.where(ok, 1, 0).astype(jnp.int32)
                    inc = plsc.cumsum(oki)
                    p = jnp.where(ok, cnt + inc - oki, CAP - LANES + lanes)
                    plsc.store_scatter(elist, [p], sv + (lo << PBITS))
                    cnt = jnp.minimum(cnt + inc[LANES - 1], CAP - 2 * GCHUNK)
                return cnt

            return lax.fori_loop(0, 32, row_body, cnt)

        def blk_body(b, cnt):
            pltpu.sync_copy(eb_h.at[pl.ds(b * 64, 64)], scan0)
            return scan_block(scan0, cnt)

        cnt = lax.fori_loop(0, 2 * SCAN_BLKS, blk_body, jnp.int32(0))
        n_pairs = (cnt + 2 * GCHUNK - 1) // (2 * GCHUNK)

        @pl.loop(0, q_chunks)
        def _pass(q):
            pltpu.sync_copy(t_h.at[pl.ds(q, 1)], tm_v.at[pl.ds(0, 1)])
            pltpu.sync_copy(m_h.at[pl.ds(q, 1)], tm_v.at[pl.ds(1, 1)])
            zf = jnp.zeros((LANES,), jnp.float32)

            @pl.loop(0, ACC_R)
            def _(r):
                for g in range(256 // LANES):
                    acc[r, pl.ds(g * LANES, LANES)] = zf

            qoff = q * N_NODES
            smask = (1 << PBITS) - 1

            def _unpack(ch, sb):
                @pl.loop(0, GCHUNK // LANES)
                def _(g):
                    sl = pl.ds(g * LANES, LANES)
                    e16 = elist[pl.ds(ch * GCHUNK + g * LANES, LANES)]
                    sb[sl] = (e16 & smask) + qoff

            def _compute(ch, rows):
                ts = [tm_v[0, pl.ds(g * LANES, LANES)] for g in range(8)]
                ms = [tm_v[1, pl.ds(g * LANES, LANES)] for g in range(8)]
                @pl.loop(0, GCHUNK // LANES)
                def _(sub):
                    lv = elist[pl.ds(ch * GCHUNK + sub * LANES, LANES)]
                    for j in range(LANES):
                        r = lv[j] >> PBITS
                        for g in range(8):
                            sl = pl.ds(g * LANES, LANES)
                            v = rows[sub * LANES + j, sl]
                            e = jnp.exp(v * ts[g] - ms[g])
                            plsc.addupdate(acc.at[r, sl], e)
                            plsc.addupdate(
                                acc.at[r, pl.ds(128 + g * LANES, LANES)],
                                e * v)

            _unpack(0, sb0)
            pltpu.async_copy(table_h.at[sb0], rows0, sem0)

            @pl.loop(0, n_pairs)
            def _(i):
                ch = i * 2
                _unpack(ch + 1, sb1)
                pltpu.async_copy(table_h.at[sb1], rows1, sem1)
                pltpu.make_async_copy(table_h.at[sb0], rows0, sem0).wait()
                _compute(ch, rows0)

                @pl.when(i < n_pairs - 1)
                def _():
                    _unpack(ch + 2, sb0)
                    pltpu.async_copy(table_h.at[sb0], rows0, sem0)

                pltpu.make_async_copy(table_h.at[sb1], rows1, sem1).wait()
                _compute(ch + 1, rows1)

            row0 = q * N_TILES * ACC_R + tid * ACC_R
            pltpu.sync_copy(acc, out_h.at[pl.ds(row0, ACC_R)])

    return k(table, edges, tq, mq)


def _sc_aggr(table, edges, t, m, q_chunks):
    o = _sc_seg_sums(table, edges, t.reshape(q_chunks, 128),
                     m.reshape(q_chunks, 128), q_chunks)
    o = o.reshape(q_chunks, N_TILES, ACC_R, 256)
    o = o[:, :, :TSLICE, :].reshape(q_chunks, N_TILES * TSLICE, 256)
    o = o[:, :N_NODES, :]
    return o[:, :, 128:], o[:, :, :128]


def _colmax_body(x_ref, t_ref, m_ref):
    i = pl.program_id(0)
    mx = jnp.max(x_ref[...] * t_ref[...], axis=0, keepdims=True)

    @pl.when(i == 0)
    def _():
        m_ref[...] = mx

    @pl.when(i > 0)
    def _():
        m_ref[...] = jnp.maximum(m_ref[...], mx)


def _colmax(x, t):
    n, d = x.shape
    return pl.pallas_call(
        _colmax_body,
        grid=(N_GRID,),
        in_specs=[
            pl.BlockSpec((ROW_BLK, d), lambda i: (i, 0)),
            pl.BlockSpec((1, d), lambda i: (0, 0)),
        ],
        out_specs=pl.BlockSpec((1, d), lambda i: (0, 0)),
        out_shape=jax.ShapeDtypeStruct((1, d), jnp.float32),
    )(x, t)


def _sage_dense_body(q_chunks, num_ref, den_ref, x_ref, wl_ref, wr_ref,
                     y_ref, s1_ref, s2_ref):
    i = pl.program_id(0)
    y = jnp.dot(x_ref[...], wr_ref[...], preferred_element_type=jnp.float32)
    for q in range(q_chunks):
        aggr = num_ref[q] / (den_ref[q] + 1e-16)
        y += jnp.dot(aggr, wl_ref[q], preferred_element_type=jnp.float32)
    y_ref[...] = y
    s1 = jnp.sum(y).reshape(1, 1)
    s2 = jnp.sum(y * y).reshape(1, 1)

    @pl.when(i == 0)
    def _():
        s1_ref[...] = s1
        s2_ref[...] = s2

    @pl.when(i > 0)
    def _():
        s1_ref[...] += s1
        s2_ref[...] += s2


def _sage_dense(num, den, x, wlt, wrt):
    n, d = x.shape
    q_chunks = num.shape[0]
    h = wrt.shape[1]
    wlq = wlt.reshape(q_chunks, 128, h)
    return pl.pallas_call(
        functools.partial(_sage_dense_body, q_chunks),
        grid=(N_GRID,),
        in_specs=[
            pl.BlockSpec((q_chunks, ROW_BLK, 128), lambda i: (0, i, 0)),
            pl.BlockSpec((q_chunks, ROW_BLK, 128), lambda i: (0, i, 0)),
            pl.BlockSpec((ROW_BLK, d), lambda i: (i, 0)),
            pl.BlockSpec((q_chunks, 128, h), lambda i: (0, 0, 0)),
            pl.BlockSpec((d, h), lambda i: (0, 0)),
        ],
        out_specs=[
            pl.BlockSpec((ROW_BLK, h), lambda i: (i, 0)),
            pl.BlockSpec((1, 1), lambda i: (0, 0)),
            pl.BlockSpec((1, 1), lambda i: (0, 0)),
        ],
        out_shape=[
            jax.ShapeDtypeStruct((n, h), jnp.float32),
            jax.ShapeDtypeStruct((1, 1), jnp.float32),
            jax.ShapeDtypeStruct((1, 1), jnp.float32),
        ],
    )(num, den, x, wlq, wrt)


def _norm_relu_body(n_elems, y_ref, s1_ref, s2_ref, w_ref, b_ref, t_ref,
                    h_ref, m_ref, c_ref):
    i = pl.program_id(0)
    mu = s1_ref[0, 0] / n_elems
    var = jnp.maximum(s2_ref[0, 0] / n_elems - mu * mu, 0.0)
    inv = 1.0 / (jnp.sqrt(var) + 1e-5)
    h = jnp.maximum((y_ref[...] - mu) * inv * w_ref[...] + b_ref[...], 0.0)
    h_ref[...] = h
    for q in range(c_ref.shape[0]):
        c_ref[q] = h[:, q * 128:(q + 1) * 128]
    mx = jnp.max(h * t_ref[...], axis=0, keepdims=True)

    @pl.when(i == 0)
    def _():
        m_ref[...] = mx

    @pl.when(i > 0)
    def _():
        m_ref[...] = jnp.maximum(m_ref[...], mx)


def _norm_relu(y, s1, s2, w, b, t):
    n, h = y.shape
    q_chunks = h // 128
    return pl.pallas_call(
        functools.partial(_norm_relu_body, float(n * h)),
        grid=(N_GRID,),
        in_specs=[
            pl.BlockSpec((ROW_BLK, h), lambda i: (i, 0)),
            pl.BlockSpec((1, 1), lambda i: (0, 0)),
            pl.BlockSpec((1, 1), lambda i: (0, 0)),
            pl.BlockSpec((1, h), lambda i: (0, 0)),
            pl.BlockSpec((1, h), lambda i: (0, 0)),
            pl.BlockSpec((1, h), lambda i: (0, 0)),
        ],
        out_specs=[
            pl.BlockSpec((ROW_BLK, h), lambda i: (i, 0)),
            pl.BlockSpec((1, h), lambda i: (0, 0)),
            pl.BlockSpec((q_chunks, ROW_BLK, 128), lambda i: (0, i, 0)),
        ],
        out_shape=[
            jax.ShapeDtypeStruct((n, h), jnp.float32),
            jax.ShapeDtypeStruct((1, h), jnp.float32),
            jax.ShapeDtypeStruct((q_chunks, n, 128), jnp.float32),
        ],
    )(y, s1, s2, w, b, t)


def _final_body(n_elems, y_ref, s1_ref, s2_ref, w_ref, b_ref, mlw_ref, fxw_ref,
                fxb_ref, out_ref, acc_ref):
    i = pl.program_id(0)
    mu = s1_ref[0, 0] / n_elems
    var = jnp.maximum(s2_ref[0, 0] / n_elems - mu * mu, 0.0)
    inv = 1.0 / (jnp.sqrt(var) + 1e-5)
    h = jnp.maximum((y_ref[...] - mu) * inv * w_ref[...] + b_ref[...], 0.0)
    cs = jnp.sum(h, axis=0, keepdims=True)

    @pl.when(i == 0)
    def _():
        acc_ref[...] = cs

    @pl.when(i > 0)
    def _():
        acc_ref[...] += cs

    @pl.when(i == pl.num_programs(0) - 1)
    def _():
        pooled = jnp.dot(acc_ref[...], mlw_ref[...],
                         preferred_element_type=jnp.float32)
        out_ref[...] = jnp.dot(pooled, fxw_ref[...],
                               preferred_element_type=jnp.float32) + fxb_ref[...]


def _final(y, s1, s2, w, b, mlwt, fxwt, fxb):
    n, h = y.shape
    return pl.pallas_call(
        functools.partial(_final_body, float(n * h)),
        grid=(N_GRID,),
        in_specs=[
            pl.BlockSpec((ROW_BLK, h), lambda i: (i, 0)),
            pl.BlockSpec((1, 1), lambda i: (0, 0)),
            pl.BlockSpec((1, 1), lambda i: (0, 0)),
            pl.BlockSpec((1, h), lambda i: (0, 0)),
            pl.BlockSpec((1, h), lambda i: (0, 0)),
            pl.BlockSpec(mlwt.shape, lambda i: (0, 0)),
            pl.BlockSpec(fxwt.shape, lambda i: (0, 0)),
            pl.BlockSpec((1, fxwt.shape[1]), lambda i: (0, 0)),
        ],
        out_specs=pl.BlockSpec((1, fxwt.shape[1]), lambda i: (0, 0)),
        out_shape=jax.ShapeDtypeStruct((1, fxwt.shape[1]), jnp.float32),
        scratch_shapes=[pltpu.VMEM((1, h), jnp.float32)],
    )(y, s1, s2, w, b, mlwt, fxwt, fxb)


def kernel(x, edge_index, t1, W1l, W1r, ln1_w, ln1_b, t2, W2l, W2r, ln2_w,
           ln2_b, mem_k, mem_conv_w, mem_lin_w, fx_w, fx_b):
    src = edge_index[0]
    dst = edge_index[1]
    pad = E_PAD - N_EDGES
    src_p = jnp.concatenate(
        [src, jnp.zeros((pad,), jnp.int32)]).reshape(SCAN_BLKS * 2, 32, 64)
    dst_p = jnp.concatenate(
        [dst, jnp.full((pad,), -1, jnp.int32)]).reshape(SCAN_BLKS * 2, 32, 64)
    edges = jnp.concatenate([src_p, dst_p], axis=1).reshape(-1, 64)

    m1 = _colmax(x, t1)
    num1, den1 = _sc_aggr(x, edges, t1, m1, 1)
    y1, s1a, s1b = _sage_dense(num1, den1, x, W1l.T, W1r.T)
    h1, m2, h1c = _norm_relu(y1, s1a, s1b, ln1_w.reshape(1, -1),
                             ln1_b.reshape(1, -1), t2)

    num2, den2 = _sc_aggr(h1c.reshape(-1, 128), edges, t2, m2, 4)
    y2, s2a, s2b = _sage_dense(num2, den2, h1, W2l.T, W2r.T)

    return _final(y2, s2a, s2b, ln2_w.reshape(1, -1), ln2_b.reshape(1, -1),
                  mem_lin_w.T, fx_w.T, fx_b.reshape(1, -1))


# speedup vs baseline: 1.4818x; 1.0337x over previous
import dataclasses
import functools

import jax
import jax.numpy as jnp
from jax import lax
from jax.experimental import pallas as pl
from jax.experimental.pallas import tpu as pltpu
from jax.experimental.pallas import tpu_sc as plsc

N_NODES = 10000
N_EDGES = 320000
ROW_BLK = 2000
N_GRID = N_NODES // ROW_BLK

LANES = 16
N_TILES = 32
E_PAD = 327680
SCAN_BLKS = 80
TSLICE = 313
ACC_R = 320
TRASH = TSLICE
CAP = 10752
GCHUNK = 80
PBITS = 14


def _sc_seg_sums(table, edges, tq, mq, q_chunks):
    mesh = plsc.VectorSubcoreMesh(core_axis_name="c", subcore_axis_name="s")
    cp = pltpu.CompilerParams()
    if "needs_layout_passes" in pltpu.CompilerParams.__dataclass_fields__:
        cp = dataclasses.replace(cp, needs_layout_passes=False)

    @functools.partial(
        pl.kernel,
        mesh=mesh,
        compiler_params=cp,
        out_type=jax.ShapeDtypeStruct((q_chunks * N_TILES * ACC_R, 256),
                                      jnp.float32),
        scratch_types=[
            pltpu.VMEM((64, 64), jnp.int32),
            pltpu.VMEM((64, 64), jnp.int32),
            pltpu.VMEM((CAP,), jnp.int32),
            pltpu.VMEM((GCHUNK,), jnp.int32),
            pltpu.VMEM((GCHUNK,), jnp.int32),
            pltpu.VMEM((GCHUNK, 128), jnp.float32),
            pltpu.VMEM((GCHUNK, 128), jnp.float32),
            pltpu.VMEM((ACC_R, 256), jnp.float32),
            pltpu.VMEM((2, 128), jnp.float32),
            pltpu.SemaphoreType.DMA,
            pltpu.SemaphoreType.DMA,
        ],
    )
    def k(table_h, eb_h, t_h, m_h, out_h,
          scan0, scan1, elist, sb0, sb1, rows0, rows1, acc, tm_v, sem0, sem1):
        c = lax.axis_index("c")
        s = lax.axis_index("s")
        tid = c * 16 + s
        base = tid * TSLICE

        tl = jnp.full((LANES,), TRASH << PBITS, jnp.int32)

        @pl.loop(0, CAP // LANES)
        def _(i):
            elist[pl.ds(i * LANES, LANES)] = tl

        lanes = lax.iota(jnp.int32, LANES)

        def scan_block(buf, cnt):
            def row_body(r, cnt):
                for gi in range(4):
                    sl = pl.ds(gi * LANES, LANES)
                    sv = buf[r, sl]
                    dv = buf[r + 32, sl]
                    lo = dv - base
                    ok = (lo >= 0) & (lo < TSLICE)
                    oki = jnp.where(ok, 1, 0).astype(jnp.int32)
                    inc = plsc.cumsum(oki)
                    p = jnp.where(ok, cnt + inc - oki, CAP - LANES + lanes)
                    plsc.store_scatter(elist, [p], sv + (lo << PBITS))
                    cnt = jnp.minimum(cnt + inc[LANES - 1], CAP - 2 * GCHUNK)
                return cnt

            return lax.fori_loop(0, 32, row_body, cnt)

        pltpu.async_copy(eb_h.at[pl.ds(0, 64)], scan0, sem0)

        def blk_pair(b, cnt):
            pltpu.async_copy(eb_h.at[pl.ds((2 * b + 1) * 64, 64)], scan1,
                             sem1)
            pltpu.make_async_copy(eb_h.at[pl.ds(0, 64)], scan0, sem0).wait()
            cnt = scan_block(scan0, cnt)

            @pl.when(b < SCAN_BLKS - 1)
            def _():
                pltpu.async_copy(eb_h.at[pl.ds((2 * b + 2) * 64, 64)], scan0,
                                 sem0)

            pltpu.make_async_copy(eb_h.at[pl.ds(0, 64)], scan1, sem1).wait()
            return scan_block(scan1, cnt)

        cnt = lax.fori_loop(0, SCAN_BLKS, blk_pair, jnp.int32(0))
        n_pairs = (cnt + 2 * GCHUNK - 1) // (2 * GCHUNK)

        @pl.loop(0, q_chunks)
        def _pass(q):
            pltpu.sync_copy(t_h.at[pl.ds(q, 1)], tm_v.at[pl.ds(0, 1)])
            pltpu.sync_copy(m_h.at[pl.ds(q, 1)], tm_v.at[pl.ds(1, 1)])
            zf = jnp.zeros((LANES,), jnp.float32)

            @pl.loop(0, ACC_R)
            def _(r):
                for g in range(256 // LANES):
                    acc[r, pl.ds(g * LANES, LANES)] = zf

            qoff = q * N_NODES
            smask = (1 << PBITS) - 1

            def _unpack(ch, sb):
                @pl.loop(0, GCHUNK // LANES)
                def _(g):
                    sl = pl.ds(g * LANES, LANES)
                    e16 = elist[pl.ds(ch * GCHUNK + g * LANES, LANES)]
                    sb[sl] = (e16 & smask) + qoff

            def _compute(ch, rows):
                ts = [tm_v[0, pl.ds(g * LANES, LANES)] for g in range(8)]
                ms = [tm_v[1, pl.ds(g * LANES, LANES)] for g in range(8)]
                @pl.loop(0, GCHUNK // LANES)
                def _(sub):
                    lv = elist[pl.ds(ch * GCHUNK + sub * LANES, LANES)]
                    for j in range(LANES):
                        r = lv[j] >> PBITS
                        for g in range(8):
                            sl = pl.ds(g * LANES, LANES)
                            v = rows[sub * LANES + j, sl]
                            e = jnp.exp(v * ts[g] - ms[g])
                            plsc.addupdate(acc.at[r, sl], e)
                            plsc.addupdate(
                                acc.at[r, pl.ds(128 + g * LANES, LANES)],
                                e * v)

            _unpack(0, sb0)
            pltpu.async_copy(table_h.at[sb0], rows0, sem0)

            @pl.loop(0, n_pairs)
            def _(i):
                ch = i * 2
                _unpack(ch + 1, sb1)
                pltpu.async_copy(table_h.at[sb1], rows1, sem1)
                pltpu.make_async_copy(table_h.at[sb0], rows0, sem0).wait()
                _compute(ch, rows0)

                @pl.when(i < n_pairs - 1)
                def _():
                    _unpack(ch + 2, sb0)
                    pltpu.async_copy(table_h.at[sb0], rows0, sem0)

                pltpu.make_async_copy(table_h.at[sb1], rows1, sem1).wait()
                _compute(ch + 1, rows1)

            row0 = q * N_TILES * ACC_R + tid * ACC_R
            pltpu.sync_copy(acc, out_h.at[pl.ds(row0, ACC_R)])

    return k(table, edges, tq, mq)


def _sc_aggr(table, edges, t, m, q_chunks):
    o = _sc_seg_sums(table, edges, t.reshape(q_chunks, 128),
                     m.reshape(q_chunks, 128), q_chunks)
    o = o.reshape(q_chunks, N_TILES, ACC_R, 256)
    o = o[:, :, :TSLICE, :].reshape(q_chunks, N_TILES * TSLICE, 256)
    o = o[:, :N_NODES, :]
    return o[:, :, 128:], o[:, :, :128]


def _colmax_body(x_ref, t_ref, m_ref):
    i = pl.program_id(0)
    mx = jnp.max(x_ref[...] * t_ref[...], axis=0, keepdims=True)

    @pl.when(i == 0)
    def _():
        m_ref[...] = mx

    @pl.when(i > 0)
    def _():
        m_ref[...] = jnp.maximum(m_ref[...], mx)


def _colmax(x, t):
    n, d = x.shape
    return pl.pallas_call(
        _colmax_body,
        grid=(N_GRID,),
        in_specs=[
            pl.BlockSpec((ROW_BLK, d), lambda i: (i, 0)),
            pl.BlockSpec((1, d), lambda i: (0, 0)),
        ],
        out_specs=pl.BlockSpec((1, d), lambda i: (0, 0)),
        out_shape=jax.ShapeDtypeStruct((1, d), jnp.float32),
    )(x, t)


def _sage_dense_body(q_chunks, num_ref, den_ref, x_ref, wl_ref, wr_ref,
                     y_ref, s1_ref, s2_ref):
    i = pl.program_id(0)
    y = jnp.dot(x_ref[...], wr_ref[...], preferred_element_type=jnp.float32)
    for q in range(q_chunks):
        aggr = num_ref[q] / (den_ref[q] + 1e-16)
        y += jnp.dot(aggr, wl_ref[q], preferred_element_type=jnp.float32)
    y_ref[...] = y
    s1 = jnp.sum(y).reshape(1, 1)
    s2 = jnp.sum(y * y).reshape(1, 1)

    @pl.when(i == 0)
    def _():
        s1_ref[...] = s1
        s2_ref[...] = s2

    @pl.when(i > 0)
    def _():
        s1_ref[...] += s1
        s2_ref[...] += s2


def _sage_dense(num, den, x, wlt, wrt):
    n, d = x.shape
    q_chunks = num.shape[0]
    h = wrt.shape[1]
    wlq = wlt.reshape(q_chunks, 128, h)
    return pl.pallas_call(
        functools.partial(_sage_dense_body, q_chunks),
        grid=(N_GRID,),
        in_specs=[
            pl.BlockSpec((q_chunks, ROW_BLK, 128), lambda i: (0, i, 0)),
            pl.BlockSpec((q_chunks, ROW_BLK, 128), lambda i: (0, i, 0)),
            pl.BlockSpec((ROW_BLK, d), lambda i: (i, 0)),
            pl.BlockSpec((q_chunks, 128, h), lambda i: (0, 0, 0)),
            pl.BlockSpec((d, h), lambda i: (0, 0)),
        ],
        out_specs=[
            pl.BlockSpec((ROW_BLK, h), lambda i: (i, 0)),
            pl.BlockSpec((1, 1), lambda i: (0, 0)),
            pl.BlockSpec((1, 1), lambda i: (0, 0)),
        ],
        out_shape=[
            jax.ShapeDtypeStruct((n, h), jnp.float32),
            jax.ShapeDtypeStruct((1, 1), jnp.float32),
            jax.ShapeDtypeStruct((1, 1), jnp.float32),
        ],
    )(num, den, x, wlq, wrt)


def _norm_relu_body(n_elems, y_ref, s1_ref, s2_ref, w_ref, b_ref, t_ref,
                    h_ref, m_ref, c_ref):
    i = pl.program_id(0)
    mu = s1_ref[0, 0] / n_elems
    var = jnp.maximum(s2_ref[0, 0] / n_elems - mu * mu, 0.0)
    inv = 1.0 / (jnp.sqrt(var) + 1e-5)
    h = jnp.maximum((y_ref[...] - mu) * inv * w_ref[...] + b_ref[...], 0.0)
    h_ref[...] = h
    for q in range(c_ref.shape[0]):
        c_ref[q] = h[:, q * 128:(q + 1) * 128]
    mx = jnp.max(h * t_ref[...], axis=0, keepdims=True)

    @pl.when(i == 0)
    def _():
        m_ref[...] = mx

    @pl.when(i > 0)
    def _():
        m_ref[...] = jnp.maximum(m_ref[...], mx)


def _norm_relu(y, s1, s2, w, b, t):
    n, h = y.shape
    q_chunks = h // 128
    return pl.pallas_call(
        functools.partial(_norm_relu_body, float(n * h)),
        grid=(N_GRID,),
        in_specs=[
            pl.BlockSpec((ROW_BLK, h), lambda i: (i, 0)),
            pl.BlockSpec((1, 1), lambda i: (0, 0)),
            pl.BlockSpec((1, 1), lambda i: (0, 0)),
            pl.BlockSpec((1, h), lambda i: (0, 0)),
            pl.BlockSpec((1, h), lambda i: (0, 0)),
            pl.BlockSpec((1, h), lambda i: (0, 0)),
        ],
        out_specs=[
            pl.BlockSpec((ROW_BLK, h), lambda i: (i, 0)),
            pl.BlockSpec((1, h), lambda i: (0, 0)),
            pl.BlockSpec((q_chunks, ROW_BLK, 128), lambda i: (0, i, 0)),
        ],
        out_shape=[
            jax.ShapeDtypeStruct((n, h), jnp.float32),
            jax.ShapeDtypeStruct((1, h), jnp.float32),
            jax.ShapeDtypeStruct((q_chunks, n, 128), jnp.float32),
        ],
    )(y, s1, s2, w, b, t)


def _final_body(n_elems, y_ref, s1_ref, s2_ref, w_ref, b_ref, mlw_ref, fxw_ref,
                fxb_ref, out_ref, acc_ref):
    i = pl.program_id(0)
    mu = s1_ref[0, 0] / n_elems
    var = jnp.maximum(s2_ref[0, 0] / n_elems - mu * mu, 0.0)
    inv = 1.0 / (jnp.sqrt(var) + 1e-5)
    h = jnp.maximum((y_ref[...] - mu) * inv * w_ref[...] + b_ref[...], 0.0)
    cs = jnp.sum(h, axis=0, keepdims=True)

    @pl.when(i == 0)
    def _():
        acc_ref[...] = cs

    @pl.when(i > 0)
    def _():
        acc_ref[...] += cs

    @pl.when(i == pl.num_programs(0) - 1)
    def _():
        pooled = jnp.dot(acc_ref[...], mlw_ref[...],
                         preferred_element_type=jnp.float32)
        out_ref[...] = jnp.dot(pooled, fxw_ref[...],
                               preferred_element_type=jnp.float32) + fxb_ref[...]


def _final(y, s1, s2, w, b, mlwt, fxwt, fxb):
    n, h = y.shape
    return pl.pallas_call(
        functools.partial(_final_body, float(n * h)),
        grid=(N_GRID,),
        in_specs=[
            pl.BlockSpec((ROW_BLK, h), lambda i: (i, 0)),
            pl.BlockSpec((1, 1), lambda i: (0, 0)),
            pl.BlockSpec((1, 1), lambda i: (0, 0)),
            pl.BlockSpec((1, h), lambda i: (0, 0)),
            pl.BlockSpec((1, h), lambda i: (0, 0)),
            pl.BlockSpec(mlwt.shape, lambda i: (0, 0)),
            pl.BlockSpec(fxwt.shape, lambda i: (0, 0)),
            pl.BlockSpec((1, fxwt.shape[1]), lambda i: (0, 0)),
        ],
        out_specs=pl.BlockSpec((1, fxwt.shape[1]), lambda i: (0, 0)),
        out_shape=jax.ShapeDtypeStruct((1, fxwt.shape[1]), jnp.float32),
        scratch_shapes=[pltpu.VMEM((1, h), jnp.float32)],
    )(y, s1, s2, w, b, mlwt, fxwt, fxb)


def kernel(x, edge_index, t1, W1l, W1r, ln1_w, ln1_b, t2, W2l, W2r, ln2_w,
           ln2_b, mem_k, mem_conv_w, mem_lin_w, fx_w, fx_b):
    src = edge_index[0]
    dst = edge_index[1]
    pad = E_PAD - N_EDGES
    src_p = jnp.concatenate(
        [src, jnp.zeros((pad,), jnp.int32)]).reshape(SCAN_BLKS * 2, 32, 64)
    dst_p = jnp.concatenate(
        [dst, jnp.full((pad,), -1, jnp.int32)]).reshape(SCAN_BLKS * 2, 32, 64)
    edges = jnp.concatenate([src_p, dst_p], axis=1).reshape(-1, 64)

    m1 = _colmax(x, t1)
    num1, den1 = _sc_aggr(x, edges, t1, m1, 1)
    y1, s1a, s1b = _sage_dense(num1, den1, x, W1l.T, W1r.T)
    h1, m2, h1c = _norm_relu(y1, s1a, s1b, ln1_w.reshape(1, -1),
                             ln1_b.reshape(1, -1), t2)

    num2, den2 = _sc_aggr(h1c.reshape(-1, 128), edges, t2, m2, 4)
    y2, s2a, s2b = _sage_dense(num2, den2, h1, W2l.T, W2r.T)

    return _final(y2, s2a, s2b, ln2_w.reshape(1, -1), ln2_b.reshape(1, -1),
                  mem_lin_w.T, fx_w.T, fx_b.reshape(1, -1))


# speedup vs baseline: 1.5385x; 1.0383x over previous
import dataclasses
import functools

import jax
import jax.numpy as jnp
from jax import lax
from jax.experimental import pallas as pl
from jax.experimental.pallas import tpu as pltpu
from jax.experimental.pallas import tpu_sc as plsc

N_NODES = 10000
N_EDGES = 320000
ROW_BLK = 2000
N_GRID = N_NODES // ROW_BLK

LANES = 16
N_TILES = 32
E_PAD = 327680
SCAN_BLKS = 80
TSLICE = 313
ACC_R = 320
TRASH = TSLICE
CAP = 10752
GCHUNK = 80
PBITS = 14


def _sc_seg_sums(table, edges, tq, mq, q_chunks, elist_in=None):
    mesh = plsc.VectorSubcoreMesh(core_axis_name="c", subcore_axis_name="s")
    cp = pltpu.CompilerParams()
    if "needs_layout_passes" in pltpu.CompilerParams.__dataclass_fields__:
        cp = dataclasses.replace(cp, needs_layout_passes=False)
    scan_mode = elist_in is None
    sums_t = jax.ShapeDtypeStruct((q_chunks * N_TILES * ACC_R, 256),
                                  jnp.float32)
    out_t = ([sums_t, jax.ShapeDtypeStruct((N_TILES * CAP,), jnp.int32),
              jax.ShapeDtypeStruct((N_TILES * LANES,), jnp.int32)]
             if scan_mode else sums_t)

    @functools.partial(
        pl.kernel,
        mesh=mesh,
        compiler_params=cp,
        out_type=out_t,
        scratch_types=[
            pltpu.VMEM((64, 64), jnp.int32),
            pltpu.VMEM((64, 64), jnp.int32),
            pltpu.VMEM((CAP,), jnp.int32),
            pltpu.VMEM((GCHUNK,), jnp.int32),
            pltpu.VMEM((GCHUNK,), jnp.int32),
            pltpu.VMEM((GCHUNK, 128), jnp.float32),
            pltpu.VMEM((GCHUNK, 128), jnp.float32),
            pltpu.VMEM((ACC_R, 256), jnp.float32),
            pltpu.VMEM((2, 128), jnp.float32),
            pltpu.SemaphoreType.DMA,
            pltpu.SemaphoreType.DMA,
        ],
    )
    def k(table_h, eb_h, t_h, m_h, *rest):
        if scan_mode:
            (out_h, el_h, cnt_h, scan0, scan1, elist, sb0, sb1, rows0, rows1,
             acc, tm_v, sem0, sem1) = rest
        else:
            (el_h, cnt_h, out_h, scan0, scan1, elist, sb0, sb1, rows0, rows1,
             acc, tm_v, sem0, sem1) = rest
        c = lax.axis_index("c")
        s = lax.axis_index("s")
        tid = c * 16 + s
        base = tid * TSLICE

        tl = jnp.full((LANES,), TRASH << PBITS, jnp.int32)

        @pl.loop(0, CAP // LANES)
        def _(i):
            elist[pl.ds(i * LANES, LANES)] = tl

        lanes = lax.iota(jnp.int32, LANES)

        def scan_block(buf, cnt):
            def row_body(r, cnt):
                for gi in range(4):
                    sl = pl.ds(gi * LANES, LANES)
                    sv = buf[r, sl]
                    dv = buf[r + 32, sl]
                    lo = dv - base
                    ok = (lo >= 0) & (lo < TSLICE)
                    oki = jnp.where(ok, 1, 0).astype(jnp.int32)
                    inc = plsc.cumsum(oki)
                    p = jnp.where(ok, cnt + inc - oki, CAP - LANES + lanes)
                    plsc.store_scatter(elist, [p], sv + (lo << PBITS))
                    cnt = jnp.minimum(cnt + inc[LANES - 1], CAP - 2 * GCHUNK)
                return cnt

            return lax.fori_loop(0, 32, row_body, cnt)

        if scan_mode:
            pltpu.async_copy(eb_h.at[pl.ds(0, 64)], scan0, sem0)

            def blk_pair(b, cnt):
                pltpu.async_copy(eb_h.at[pl.ds((2 * b + 1) * 64, 64)], scan1,
                                 sem1)
                pltpu.make_async_copy(eb_h.at[pl.ds(0, 64)], scan0,
                                      sem0).wait()
                cnt = scan_block(scan0, cnt)

                @pl.when(b < SCAN_BLKS - 1)
                def _():
                    pltpu.async_copy(eb_h.at[pl.ds((2 * b + 2) * 64, 64)],
                                     scan0, sem0)

                pltpu.make_async_copy(eb_h.at[pl.ds(0, 64)], scan1,
                                      sem1).wait()
                return scan_block(scan1, cnt)

            cnt = lax.fori_loop(0, SCAN_BLKS, blk_pair, jnp.int32(0))
            pltpu.sync_copy(elist, el_h.at[pl.ds(tid * CAP, CAP)])
            sb0[pl.ds(0, LANES)] = jnp.full((LANES,), 0, jnp.int32) + cnt
            pltpu.sync_copy(sb0.at[pl.ds(0, LANES)],
                            cnt_h.at[pl.ds(tid * LANES, LANES)])
        else:
            pltpu.sync_copy(el_h.at[pl.ds(tid * CAP, CAP)], elist)
            pltpu.sync_copy(cnt_h.at[pl.ds(tid * LANES, LANES)],
                            sb0.at[pl.ds(0, LANES)])
            cv = sb0[pl.ds(0, LANES)]
            cnt = cv[0]
        n_pairs = (cnt + 2 * GCHUNK - 1) // (2 * GCHUNK)

        @pl.loop(0, q_chunks)
        def _pass(q):
            pltpu.sync_copy(t_h.at[pl.ds(q, 1)], tm_v.at[pl.ds(0, 1)])
            pltpu.sync_copy(m_h.at[pl.ds(q, 1)], tm_v.at[pl.ds(1, 1)])
            zf = jnp.zeros((LANES,), jnp.float32)

            @pl.loop(0, ACC_R)
            def _(r):
                for g in range(256 // LANES):
                    acc[r, pl.ds(g * LANES, LANES)] = zf

            qoff = q * N_NODES
            smask = (1 << PBITS) - 1

            def _unpack(ch, sb):
                @pl.loop(0, GCHUNK // LANES)
                def _(g):
                    sl = pl.ds(g * LANES, LANES)
                    e16 = elist[pl.ds(ch * GCHUNK + g * LANES, LANES)]
                    sb[sl] = (e16 & smask) + qoff

            def _compute(ch, rows):
                ts = [tm_v[0, pl.ds(g * LANES, LANES)] for g in range(8)]
                ms = [tm_v[1, pl.ds(g * LANES, LANES)] for g in range(8)]
                @pl.loop(0, GCHUNK // LANES)
                def _(sub):
                    lv = elist[pl.ds(ch * GCHUNK + sub * LANES, LANES)]
                    for j in range(LANES):
                        r = lv[j] >> PBITS
                        for g in range(8):
                            sl = pl.ds(g * LANES, LANES)
                            v = rows[sub * LANES + j, sl]
                            e = jnp.exp(v * ts[g] - ms[g])
                            plsc.addupdate(acc.at[r, sl], e)
                            plsc.addupdate(
                                acc.at[r, pl.ds(128 + g * LANES, LANES)],
                                e * v)

            _unpack(0, sb0)
            pltpu.async_copy(table_h.at[sb0], rows0, sem0)

            @pl.loop(0, n_pairs)
            def _(i):
                ch = i * 2
                _unpack(ch + 1, sb1)
                pltpu.async_copy(table_h.at[sb1], rows1, sem1)
                pltpu.make_async_copy(table_h.at[sb0], rows0, sem0).wait()
                _compute(ch, rows0)

                @pl.when(i < n_pairs - 1)
                def _():
                    _unpack(ch + 2, sb0)
                    pltpu.async_copy(table_h.at[sb0], rows0, sem0)

                pltpu.make_async_copy(table_h.at[sb1], rows1, sem1).wait()
                _compute(ch + 1, rows1)

            row0 = q * N_TILES * ACC_R + tid * ACC_R
            pltpu.sync_copy(acc, out_h.at[pl.ds(row0, ACC_R)])

    if scan_mode:
        return k(table, edges, tq, mq)
    return k(table, edges, tq, mq, *elist_in)


def _sc_aggr(table, edges, t, m, q_chunks, elist_in=None):
    o = _sc_seg_sums(table, edges, t.reshape(q_chunks, 128),
                     m.reshape(q_chunks, 128), q_chunks, elist_in)
    lists = None
    if elist_in is None:
        o, el, cn = o
        lists = (el, cn)
    o = o.reshape(q_chunks, N_TILES, ACC_R, 256)
    o = o[:, :, :TSLICE, :].reshape(q_chunks, N_TILES * TSLICE, 256)
    o = o[:, :N_NODES, :]
    return o[:, :, 128:], o[:, :, :128], lists


def _colmax_body(x_ref, t_ref, m_ref):
    i = pl.program_id(0)
    mx = jnp.max(x_ref[...] * t_ref[...], axis=0, keepdims=True)

    @pl.when(i == 0)
    def _():
        m_ref[...] = mx

    @pl.when(i > 0)
    def _():
        m_ref[...] = jnp.maximum(m_ref[...], mx)


def _colmax(x, t):
    n, d = x.shape
    return pl.pallas_call(
        _colmax_body,
        grid=(N_GRID,),
        in_specs=[
            pl.BlockSpec((ROW_BLK, d), lambda i: (i, 0)),
            pl.BlockSpec((1, d), lambda i: (0, 0)),
        ],
        out_specs=pl.BlockSpec((1, d), lambda i: (0, 0)),
        out_shape=jax.ShapeDtypeStruct((1, d), jnp.float32),
    )(x, t)


def _sage_dense_body(q_chunks, num_ref, den_ref, x_ref, wl_ref, wr_ref,
                     y_ref, s1_ref, s2_ref):
    i = pl.program_id(0)
    y = jnp.dot(x_ref[...], wr_ref[...], preferred_element_type=jnp.float32)
    for q in range(q_chunks):
        aggr = num_ref[q] / (den_ref[q] + 1e-16)
        y += jnp.dot(aggr, wl_ref[q], preferred_element_type=jnp.float32)
    y_ref[...] = y
    s1 = jnp.sum(y).reshape(1, 1)
    s2 = jnp.sum(y * y).reshape(1, 1)

    @pl.when(i == 0)
    def _():
        s1_ref[...] = s1
        s2_ref[...] = s2

    @pl.when(i > 0)
    def _():
        s1_ref[...] += s1
        s2_ref[...] += s2


def _sage_dense(num, den, x, wlt, wrt):
    n, d = x.shape
    q_chunks = num.shape[0]
    h = wrt.shape[1]
    wlq = wlt.reshape(q_chunks, 128, h)
    return pl.pallas_call(
        functools.partial(_sage_dense_body, q_chunks),
        grid=(N_GRID,),
        in_specs=[
            pl.BlockSpec((q_chunks, ROW_BLK, 128), lambda i: (0, i, 0)),
            pl.BlockSpec((q_chunks, ROW_BLK, 128), lambda i: (0, i, 0)),
            pl.BlockSpec((ROW_BLK, d), lambda i: (i, 0)),
            pl.BlockSpec((q_chunks, 128, h), lambda i: (0, 0, 0)),
            pl.BlockSpec((d, h), lambda i: (0, 0)),
        ],
        out_specs=[
            pl.BlockSpec((ROW_BLK, h), lambda i: (i, 0)),
            pl.BlockSpec((1, 1), lambda i: (0, 0)),
            pl.BlockSpec((1, 1), lambda i: (0, 0)),
        ],
        out_shape=[
            jax.ShapeDtypeStruct((n, h), jnp.float32),
            jax.ShapeDtypeStruct((1, 1), jnp.float32),
            jax.ShapeDtypeStruct((1, 1), jnp.float32),
        ],
    )(num, den, x, wlq, wrt)


def _norm_relu_body(n_elems, y_ref, s1_ref, s2_ref, w_ref, b_ref, t_ref,
                    h_ref, m_ref, c_ref):
    i = pl.program_id(0)
    mu = s1_ref[0, 0] / n_elems
    var = jnp.maximum(s2_ref[0, 0] / n_elems - mu * mu, 0.0)
    inv = 1.0 / (jnp.sqrt(var) + 1e-5)
    h = jnp.maximum((y_ref[...] - mu) * inv * w_ref[...] + b_ref[...], 0.0)
    h_ref[...] = h
    for q in range(c_ref.shape[0]):
        c_ref[q] = h[:, q * 128:(q + 1) * 128]
    mx = jnp.max(h * t_ref[...], axis=0, keepdims=True)

    @pl.when(i == 0)
    def _():
        m_ref[...] = mx

    @pl.when(i > 0)
    def _():
        m_ref[...] = jnp.maximum(m_ref[...], mx)


def _norm_relu(y, s1, s2, w, b, t):
    n, h = y.shape
    q_chunks = h // 128
    return pl.pallas_call(
        functools.partial(_norm_relu_body, float(n * h)),
        grid=(N_GRID,),
        in_specs=[
            pl.BlockSpec((ROW_BLK, h), lambda i: (i, 0)),
            pl.BlockSpec((1, 1), lambda i: (0, 0)),
            pl.BlockSpec((1, 1), lambda i: (0, 0)),
            pl.BlockSpec((1, h), lambda i: (0, 0)),
            pl.BlockSpec((1, h), lambda i: (0, 0)),
            pl.BlockSpec((1, h), lambda i: (0, 0)),
        ],
        out_specs=[
            pl.BlockSpec((ROW_BLK, h), lambda i: (i, 0)),
            pl.BlockSpec((1, h), lambda i: (0, 0)),
            pl.BlockSpec((q_chunks, ROW_BLK, 128), lambda i: (0, i, 0)),
        ],
        out_shape=[
            jax.ShapeDtypeStruct((n, h), jnp.float32),
            jax.ShapeDtypeStruct((1, h), jnp.float32),
            jax.ShapeDtypeStruct((q_chunks, n, 128), jnp.float32),
        ],
    )(y, s1, s2, w, b, t)


def _final_body(n_elems, y_ref, s1_ref, s2_ref, w_ref, b_ref, mlw_ref, fxw_ref,
                fxb_ref, out_ref, acc_ref):
    i = pl.program_id(0)
    mu = s1_ref[0, 0] / n_elems
    var = jnp.maximum(s2_ref[0, 0] / n_elems - mu * mu, 0.0)
    inv = 1.0 / (jnp.sqrt(var) + 1e-5)
    h = jnp.maximum((y_ref[...] - mu) * inv * w_ref[...] + b_ref[...], 0.0)
    cs = jnp.sum(h, axis=0, keepdims=True)

    @pl.when(i == 0)
    def _():
        acc_ref[...] = cs

    @pl.when(i > 0)
    def _():
        acc_ref[...] += cs

    @pl.when(i == pl.num_programs(0) - 1)
    def _():
        pooled = jnp.dot(acc_ref[...], mlw_ref[...],
                         preferred_element_type=jnp.float32)
        out_ref[...] = jnp.dot(pooled, fxw_ref[...],
                               preferred_element_type=jnp.float32) + fxb_ref[...]


def _final(y, s1, s2, w, b, mlwt, fxwt, fxb):
    n, h = y.shape
    return pl.pallas_call(
        functools.partial(_final_body, float(n * h)),
        grid=(N_GRID,),
        in_specs=[
            pl.BlockSpec((ROW_BLK, h), lambda i: (i, 0)),
            pl.BlockSpec((1, 1), lambda i: (0, 0)),
            pl.BlockSpec((1, 1), lambda i: (0, 0)),
            pl.BlockSpec((1, h), lambda i: (0, 0)),
            pl.BlockSpec((1, h), lambda i: (0, 0)),
            pl.BlockSpec(mlwt.shape, lambda i: (0, 0)),
            pl.BlockSpec(fxwt.shape, lambda i: (0, 0)),
            pl.BlockSpec((1, fxwt.shape[1]), lambda i: (0, 0)),
        ],
        out_specs=pl.BlockSpec((1, fxwt.shape[1]), lambda i: (0, 0)),
        out_shape=jax.ShapeDtypeStruct((1, fxwt.shape[1]), jnp.float32),
        scratch_shapes=[pltpu.VMEM((1, h), jnp.float32)],
    )(y, s1, s2, w, b, mlwt, fxwt, fxb)


def kernel(x, edge_index, t1, W1l, W1r, ln1_w, ln1_b, t2, W2l, W2r, ln2_w,
           ln2_b, mem_k, mem_conv_w, mem_lin_w, fx_w, fx_b):
    src = edge_index[0]
    dst = edge_index[1]
    pad = E_PAD - N_EDGES
    src_p = jnp.concatenate(
        [src, jnp.zeros((pad,), jnp.int32)]).reshape(SCAN_BLKS * 2, 32, 64)
    dst_p = jnp.concatenate(
        [dst, jnp.full((pad,), -1, jnp.int32)]).reshape(SCAN_BLKS * 2, 32, 64)
    edges = jnp.concatenate([src_p, dst_p], axis=1).reshape(-1, 64)

    m1 = _colmax(x, t1)
    num1, den1, lists = _sc_aggr(x, edges, t1, m1, 1)
    y1, s1a, s1b = _sage_dense(num1, den1, x, W1l.T, W1r.T)
    h1, m2, h1c = _norm_relu(y1, s1a, s1b, ln1_w.reshape(1, -1),
                             ln1_b.reshape(1, -1), t2)

    num2, den2, _ = _sc_aggr(h1c.reshape(-1, 128), edges, t2, m2, 4,
                             elist_in=lists)
    y2, s2a, s2b = _sage_dense(num2, den2, h1, W2l.T, W2r.T)

    return _final(y2, s2a, s2b, ln2_w.reshape(1, -1), ln2_b.reshape(1, -1),
                  mem_lin_w.T, fx_w.T, fx_b.reshape(1, -1))


# speedup vs baseline: 1.5412x; 1.0018x over previous
import dataclasses
import functools

import jax
import jax.numpy as jnp
from jax import lax
from jax.experimental import pallas as pl
from jax.experimental.pallas import tpu as pltpu
from jax.experimental.pallas import tpu_sc as plsc

N_NODES = 10000
N_EDGES = 320000
ROW_BLK = 2000
N_GRID = N_NODES // ROW_BLK

LANES = 16
N_TILES = 32
E_PAD = 327680
SCAN_BLKS = 80
TSLICE = 313
ACC_R = 320
TRASH = TSLICE
CAP = 10752
GCHUNK = 80
PBITS = 14


def _sc_seg_sums(table, edges, tq, mq, q_chunks, elist_in=None, gchunk=80):
    mesh = plsc.VectorSubcoreMesh(core_axis_name="c", subcore_axis_name="s")
    cp = pltpu.CompilerParams()
    if "needs_layout_passes" in pltpu.CompilerParams.__dataclass_fields__:
        cp = dataclasses.replace(cp, needs_layout_passes=False)
    scan_mode = elist_in is None
    G = gchunk
    sums_t = jax.ShapeDtypeStruct((q_chunks * N_TILES * ACC_R, 256),
                                  jnp.float32)
    out_t = ([sums_t, jax.ShapeDtypeStruct((N_TILES * CAP,), jnp.int32),
              jax.ShapeDtypeStruct((N_TILES * LANES,), jnp.int32)]
             if scan_mode else sums_t)
    scratch = ([pltpu.VMEM((64, 64), jnp.int32)] * 2 if scan_mode else []) + [
        pltpu.VMEM((CAP,), jnp.int32),
        pltpu.VMEM((G,), jnp.int32),
        pltpu.VMEM((G,), jnp.int32),
        pltpu.VMEM((G, 128), jnp.float32),
        pltpu.VMEM((G, 128), jnp.float32),
        pltpu.VMEM((ACC_R, 256), jnp.float32),
        pltpu.VMEM((2, 128), jnp.float32),
        pltpu.SemaphoreType.DMA,
        pltpu.SemaphoreType.DMA,
    ]

    @functools.partial(
        pl.kernel,
        mesh=mesh,
        compiler_params=cp,
        out_type=out_t,
        scratch_types=scratch,
    )
    def k(table_h, eb_h, t_h, m_h, *rest):
        if scan_mode:
            (out_h, el_h, cnt_h, scan0, scan1, elist, sb0, sb1, rows0, rows1,
             acc, tm_v, sem0, sem1) = rest
        else:
            (el_h, cnt_h, out_h, elist, sb0, sb1, rows0, rows1,
             acc, tm_v, sem0, sem1) = rest
        c = lax.axis_index("c")
        s = lax.axis_index("s")
        tid = c * 16 + s
        base = tid * TSLICE

        if scan_mode:
            tl = jnp.full((LANES,), TRASH << PBITS, jnp.int32)

            @pl.loop(0, CAP // LANES)
            def _(i):
                elist[pl.ds(i * LANES, LANES)] = tl

            lanes = lax.iota(jnp.int32, LANES)

            def scan_block(buf, cnt):
                def row_body(r, cnt):
                    for gi in range(4):
                        sl = pl.ds(gi * LANES, LANES)
                        sv = buf[r, sl]
                        dv = buf[r + 32, sl]
                        lo = dv - base
                        ok = (lo >= 0) & (lo < TSLICE)
                        oki = jnp.where(ok, 1, 0).astype(jnp.int32)
                        inc = plsc.cumsum(oki)
                        p = jnp.where(ok, cnt + inc - oki,
                                      CAP - LANES + lanes)
                        plsc.store_scatter(elist, [p], sv + (lo << PBITS))
                        cnt = jnp.minimum(cnt + inc[LANES - 1], CAP - 256)
                    return cnt

                return lax.fori_loop(0, 32, row_body, cnt)

            pltpu.async_copy(eb_h.at[pl.ds(0, 64)], scan0, sem0)

            def blk_pair(b, cnt):
                pltpu.async_copy(eb_h.at[pl.ds((2 * b + 1) * 64, 64)], scan1,
                                 sem1)
                pltpu.make_async_copy(eb_h.at[pl.ds(0, 64)], scan0,
                                      sem0).wait()
                cnt = scan_block(scan0, cnt)

                @pl.when(b < SCAN_BLKS - 1)
                def _():
                    pltpu.async_copy(eb_h.at[pl.ds((2 * b + 2) * 64, 64)],
                                     scan0, sem0)

                pltpu.make_async_copy(eb_h.at[pl.ds(0, 64)], scan1,
                                      sem1).wait()
                return scan_block(scan1, cnt)

            cnt = lax.fori_loop(0, SCAN_BLKS, blk_pair, jnp.int32(0))
            pltpu.sync_copy(elist, el_h.at[pl.ds(tid * CAP, CAP)])
            sb0[pl.ds(0, LANES)] = jnp.full((LANES,), 0, jnp.int32) + cnt
            pltpu.sync_copy(sb0.at[pl.ds(0, LANES)],
                            cnt_h.at[pl.ds(tid * LANES, LANES)])
        else:
            pltpu.sync_copy(el_h.at[pl.ds(tid * CAP, CAP)], elist)
            pltpu.sync_copy(cnt_h.at[pl.ds(tid * LANES, LANES)],
                            sb0.at[pl.ds(0, LANES)])
            cv = sb0[pl.ds(0, LANES)]
            cnt = cv[0]
        n_pairs = (cnt + 2 * G - 1) // (2 * G)

        @pl.loop(0, q_chunks)
        def _pass(q):
            pltpu.sync_copy(t_h.at[pl.ds(q, 1)], tm_v.at[pl.ds(0, 1)])
            pltpu.sync_copy(m_h.at[pl.ds(q, 1)], tm_v.at[pl.ds(1, 1)])
            zf = jnp.zeros((LANES,), jnp.float32)

            @pl.loop(0, ACC_R)
            def _(r):
                for g in range(256 // LANES):
                    acc[r, pl.ds(g * LANES, LANES)] = zf

            qoff = q * N_NODES
            smask = (1 << PBITS) - 1

            def _unpack(ch, sb):
                @pl.loop(0, G // LANES)
                def _(g):
                    sl = pl.ds(g * LANES, LANES)
                    e16 = elist[pl.ds(ch * G + g * LANES, LANES)]
                    sb[sl] = (e16 & smask) + qoff

            def _compute(ch, rows):
                ts = [tm_v[0, pl.ds(g * LANES, LANES)] for g in range(8)]
                ms = [tm_v[1, pl.ds(g * LANES, LANES)] for g in range(8)]

                @pl.loop(0, G // LANES)
                def _(sub):
                    lv = elist[pl.ds(ch * G + sub * LANES, LANES)]
                    for j in range(LANES):
                        r = lv[j] >> PBITS
                        for g in range(8):
                            sl = pl.ds(g * LANES, LANES)
                            v = rows[sub * LANES + j, sl]
                            e = jnp.exp(v * ts[g] - ms[g])
                            plsc.addupdate(acc.at[r, sl], e)
                            plsc.addupdate(
                                acc.at[r, pl.ds(128 + g * LANES, LANES)],
                                e * v)

            _unpack(0, sb0)
            pltpu.async_copy(table_h.at[sb0], rows0, sem0)

            @pl.loop(0, n_pairs)
            def _(i):
                ch = i * 2
                _unpack(ch + 1, sb1)
                pltpu.async_copy(table_h.at[sb1], rows1, sem1)
                pltpu.make_async_copy(table_h.at[sb0], rows0, sem0).wait()
                _compute(ch, rows0)

                @pl.when(i < n_pairs - 1)
                def _():
                    _unpack(ch + 2, sb0)
                    pltpu.async_copy(table_h.at[sb0], rows0, sem0)

                pltpu.make_async_copy(table_h.at[sb1], rows1, sem1).wait()
                _compute(ch + 1, rows1)

            row0 = q * N_TILES * ACC_R + tid * ACC_R
            pltpu.sync_copy(acc, out_h.at[pl.ds(row0, ACC_R)])

    if scan_mode:
        return k(table, edges, tq, mq)
    return k(table, edges, tq, mq, *elist_in)


def _sc_aggr(table, edges, t, m, q_chunks, elist_in=None, gchunk=80):
    o = _sc_seg_sums(table, edges, t.reshape(q_chunks, 128),
                     m.reshape(q_chunks, 128), q_chunks, elist_in, gchunk)
    lists = None
    if elist_in is None:
        o, el, cn = o
        lists = (el, cn)
    o = o.reshape(q_chunks, N_TILES, ACC_R, 256)
    o = o[:, :, :TSLICE, :].reshape(q_chunks, N_TILES * TSLICE, 256)
    o = o[:, :N_NODES, :]
    return o[:, :, 128:], o[:, :, :128], lists


def _colmax_body(x_ref, t_ref, m_ref):
    i = pl.program_id(0)
    mx = jnp.max(x_ref[...] * t_ref[...], axis=0, keepdims=True)

    @pl.when(i == 0)
    def _():
        m_ref[...] = mx

    @pl.when(i > 0)
    def _():
        m_ref[...] = jnp.maximum(m_ref[...], mx)


def _colmax(x, t):
    n, d = x.shape
    return pl.pallas_call(
        _colmax_body,
        grid=(N_GRID,),
        in_specs=[
            pl.BlockSpec((ROW_BLK, d), lambda i: (i, 0)),
            pl.BlockSpec((1, d), lambda i: (0, 0)),
        ],
        out_specs=pl.BlockSpec((1, d), lambda i: (0, 0)),
        out_shape=jax.ShapeDtypeStruct((1, d), jnp.float32),
    )(x, t)


def _sage_dense_body(q_chunks, num_ref, den_ref, x_ref, wl_ref, wr_ref,
                     y_ref, s1_ref, s2_ref):
    i = pl.program_id(0)
    y = jnp.dot(x_ref[...], wr_ref[...], preferred_element_type=jnp.float32)
    for q in range(q_chunks):
        aggr = num_ref[q] / (den_ref[q] + 1e-16)
        y += jnp.dot(aggr, wl_ref[q], preferred_element_type=jnp.float32)
    y_ref[...] = y
    s1 = jnp.sum(y).reshape(1, 1)
    s2 = jnp.sum(y * y).reshape(1, 1)

    @pl.when(i == 0)
    def _():
        s1_ref[...] = s1
        s2_ref[...] = s2

    @pl.when(i > 0)
    def _():
        s1_ref[...] += s1
        s2_ref[...] += s2


def _sage_dense(num, den, x, wlt, wrt):
    n, d = x.shape
    q_chunks = num.shape[0]
    h = wrt.shape[1]
    wlq = wlt.reshape(q_chunks, 128, h)
    return pl.pallas_call(
        functools.partial(_sage_dense_body, q_chunks),
        grid=(N_GRID,),
        in_specs=[
            pl.BlockSpec((q_chunks, ROW_BLK, 128), lambda i: (0, i, 0)),
            pl.BlockSpec((q_chunks, ROW_BLK, 128), lambda i: (0, i, 0)),
            pl.BlockSpec((ROW_BLK, d), lambda i: (i, 0)),
            pl.BlockSpec((q_chunks, 128, h), lambda i: (0, 0, 0)),
            pl.BlockSpec((d, h), lambda i: (0, 0)),
        ],
        out_specs=[
            pl.BlockSpec((ROW_BLK, h), lambda i: (i, 0)),
            pl.BlockSpec((1, 1), lambda i: (0, 0)),
            pl.BlockSpec((1, 1), lambda i: (0, 0)),
        ],
        out_shape=[
            jax.ShapeDtypeStruct((n, h), jnp.float32),
            jax.ShapeDtypeStruct((1, 1), jnp.float32),
            jax.ShapeDtypeStruct((1, 1), jnp.float32),
        ],
    )(num, den, x, wlq, wrt)


def _norm_relu_body(n_elems, y_ref, s1_ref, s2_ref, w_ref, b_ref, t_ref,
                    h_ref, m_ref, c_ref):
    i = pl.program_id(0)
    mu = s1_ref[0, 0] / n_elems
    var = jnp.maximum(s2_ref[0, 0] / n_elems - mu * mu, 0.0)
    inv = 1.0 / (jnp.sqrt(var) + 1e-5)
    h = jnp.maximum((y_ref[...] - mu) * inv * w_ref[...] + b_ref[...], 0.0)
    h_ref[...] = h
    for q in range(c_ref.shape[0]):
        c_ref[q] = h[:, q * 128:(q + 1) * 128]
    mx = jnp.max(h * t_ref[...], axis=0, keepdims=True)

    @pl.when(i == 0)
    def _():
        m_ref[...] = mx

    @pl.when(i > 0)
    def _():
        m_ref[...] = jnp.maximum(m_ref[...], mx)


def _norm_relu(y, s1, s2, w, b, t):
    n, h = y.shape
    q_chunks = h // 128
    return pl.pallas_call(
        functools.partial(_norm_relu_body, float(n * h)),
        grid=(N_GRID,),
        in_specs=[
            pl.BlockSpec((ROW_BLK, h), lambda i: (i, 0)),
            pl.BlockSpec((1, 1), lambda i: (0, 0)),
            pl.BlockSpec((1, 1), lambda i: (0, 0)),
            pl.BlockSpec((1, h), lambda i: (0, 0)),
            pl.BlockSpec((1, h), lambda i: (0, 0)),
            pl.BlockSpec((1, h), lambda i: (0, 0)),
        ],
        out_specs=[
            pl.BlockSpec((ROW_BLK, h), lambda i: (i, 0)),
            pl.BlockSpec((1, h), lambda i: (0, 0)),
            pl.BlockSpec((q_chunks, ROW_BLK, 128), lambda i: (0, i, 0)),
        ],
        out_shape=[
            jax.ShapeDtypeStruct((n, h), jnp.float32),
            jax.ShapeDtypeStruct((1, h), jnp.float32),
            jax.ShapeDtypeStruct((q_chunks, n, 128), jnp.float32),
        ],
    )(y, s1, s2, w, b, t)


def _final_body(n_elems, y_ref, s1_ref, s2_ref, w_ref, b_ref, mlw_ref, fxw_ref,
                fxb_ref, out_ref, acc_ref):
    i = pl.program_id(0)
    mu = s1_ref[0, 0] / n_elems
    var = jnp.maximum(s2_ref[0, 0] / n_elems - mu * mu, 0.0)
    inv = 1.0 / (jnp.sqrt(var) + 1e-5)
    h = jnp.maximum((y_ref[...] - mu) * inv * w_ref[...] + b_ref[...], 0.0)
    cs = jnp.sum(h, axis=0, keepdims=True)

    @pl.when(i == 0)
    def _():
        acc_ref[...] = cs

    @pl.when(i > 0)
    def _():
        acc_ref[...] += cs

    @pl.when(i == pl.num_programs(0) - 1)
    def _():
        pooled = jnp.dot(acc_ref[...], mlw_ref[...],
                         preferred_element_type=jnp.float32)
        out_ref[...] = jnp.dot(pooled, fxw_ref[...],
                               preferred_element_type=jnp.float32) + fxb_ref[...]


def _final(y, s1, s2, w, b, mlwt, fxwt, fxb):
    n, h = y.shape
    return pl.pallas_call(
        functools.partial(_final_body, float(n * h)),
        grid=(N_GRID,),
        in_specs=[
            pl.BlockSpec((ROW_BLK, h), lambda i: (i, 0)),
            pl.BlockSpec((1, 1), lambda i: (0, 0)),
            pl.BlockSpec((1, 1), lambda i: (0, 0)),
            pl.BlockSpec((1, h), lambda i: (0, 0)),
            pl.BlockSpec((1, h), lambda i: (0, 0)),
            pl.BlockSpec(mlwt.shape, lambda i: (0, 0)),
            pl.BlockSpec(fxwt.shape, lambda i: (0, 0)),
            pl.BlockSpec((1, fxwt.shape[1]), lambda i: (0, 0)),
        ],
        out_specs=pl.BlockSpec((1, fxwt.shape[1]), lambda i: (0, 0)),
        out_shape=jax.ShapeDtypeStruct((1, fxwt.shape[1]), jnp.float32),
        scratch_shapes=[pltpu.VMEM((1, h), jnp.float32)],
    )(y, s1, s2, w, b, mlwt, fxwt, fxb)


def kernel(x, edge_index, t1, W1l, W1r, ln1_w, ln1_b, t2, W2l, W2r, ln2_w,
           ln2_b, mem_k, mem_conv_w, mem_lin_w, fx_w, fx_b):
    src = edge_index[0]
    dst = edge_index[1]
    pad = E_PAD - N_EDGES
    src_p = jnp.concatenate(
        [src, jnp.zeros((pad,), jnp.int32)]).reshape(SCAN_BLKS * 2, 32, 64)
    dst_p = jnp.concatenate(
        [dst, jnp.full((pad,), -1, jnp.int32)]).reshape(SCAN_BLKS * 2, 32, 64)
    edges = jnp.concatenate([src_p, dst_p], axis=1).reshape(-1, 64)

    m1 = _colmax(x, t1)
    num1, den1, lists = _sc_aggr(x, edges, t1, m1, 1)
    y1, s1a, s1b = _sage_dense(num1, den1, x, W1l.T, W1r.T)
    h1, m2, h1c = _norm_relu(y1, s1a, s1b, ln1_w.reshape(1, -1),
                             ln1_b.reshape(1, -1), t2)

    num2, den2, _ = _sc_aggr(h1c.reshape(-1, 128), edges, t2, m2, 4,
                             elist_in=lists, gchunk=112)
    y2, s2a, s2b = _sage_dense(num2, den2, h1, W2l.T, W2r.T)

    return _final(y2, s2a, s2b, ln2_w.reshape(1, -1), ln2_b.reshape(1, -1),
                  mem_lin_w.T, fx_w.T, fx_b.reshape(1, -1))
